# baseline (speedup 1.0000x reference)
import functools

import jax
import jax.numpy as jnp
from jax import lax
from jax.experimental import pallas as pl
from jax.experimental.pallas import tpu as pltpu
from jax.experimental.pallas import tpu_sc as plsc

_L = 16


def _sigmoid(x):
    return 1.0 / (1.0 + jnp.exp(-x))


@functools.partial(jax.jit, static_argnums=(3, 4, 5))
def _run(params, idx_t, dns_t, num_workers, vpad, ndense):
    num_fields, B = idx_t.shape
    bpw = B // num_workers
    ngroups = bpw // _L
    psize = params.shape[0]
    mesh = plsc.VectorSubcoreMesh(core_axis_name="c", subcore_axis_name="s")

    @functools.partial(
        pl.kernel,
        mesh=mesh,
        out_type=jax.ShapeDtypeStruct((B,), jnp.float32),
        scratch_types=[
            pltpu.VMEM((psize,), jnp.float32),
            pltpu.VMEM((num_fields, bpw), jnp.int32),
            pltpu.VMEM((ndense, bpw), jnp.float32),
            pltpu.VMEM((bpw,), jnp.float32),
            pltpu.SemaphoreType.DMA,
        ],
        compiler_params=pltpu.CompilerParams(
            needs_layout_passes=False, use_tc_tiling_on_sc=True
        ),
    )
    def k(params_hbm, idx_hbm, dns_hbm, out_hbm,
          params_v, idx_v, dns_v, out_v, sem):
        wid = lax.axis_index("s") * 2 + lax.axis_index("c")
        base = wid * bpw
        c1 = pltpu.async_copy(params_hbm, params_v, sem)
        c2 = pltpu.async_copy(idx_hbm.at[:, pl.ds(base, bpw)], idx_v, sem)
        c3 = pltpu.async_copy(dns_hbm.at[:, pl.ds(base, bpw)], dns_v, sem)
        c1.wait()
        c2.wait()
        c3.wait()

        lane = lax.iota(jnp.int32, _L)
        zero = jnp.zeros((_L,), jnp.int32)

        @plsc.parallel_loop(0, ngroups, 1, unroll=4)
        def body(g):
            rows = g * _L + lane
            acc = params_v[pl.ds(vpad + ndense * _L, _L)]
            for j in range(ndense):
                dv = plsc.load_gather(dns_v, [zero + j, rows])
                acc = acc + dv * params_v[pl.ds(vpad + j * _L, _L)]
            for f in range(num_fields):
                ii = plsc.load_gather(idx_v, [zero + f, rows])
                acc = acc + plsc.load_gather(params_v, [ii])
            out_v[pl.ds(g * _L, _L)] = _sigmoid(acc)
        pltpu.sync_copy(out_v, out_hbm.at[pl.ds(base, bpw)])

    return k(params, idx_t, dns_t)


def kernel(dense_input, sparse_input, emb_table, fm_W, fm_b):
    B, ndense = dense_input.shape
    V = emb_table.shape[0]
    NW = 32

    idx_t = sparse_input.astype(jnp.int32).T
    dns_t = dense_input.astype(jnp.float32).T
    vpad = ((V + 127) // 128) * 128
    params = jnp.concatenate([
        emb_table.astype(jnp.float32).reshape(V),
        jnp.zeros((vpad - V,), jnp.float32),
        jnp.broadcast_to(
            fm_W.astype(jnp.float32).reshape(ndense, 1), (ndense, _L)
        ).reshape(ndense * _L),
        jnp.broadcast_to(fm_b.astype(jnp.float32).reshape(1), (_L,)),
    ])

    out = _run(params, idx_t, dns_t, NW, vpad, ndense)
    return out.reshape(B, 1)


# speedup vs baseline: 1.0220x; 1.0220x over previous
import functools

import jax
import jax.numpy as jnp
from jax import lax
from jax.experimental import pallas as pl
from jax.experimental.pallas import tpu as pltpu
from jax.experimental.pallas import tpu_sc as plsc

_L = 16


def _sigmoid(x):
    return 1.0 / (1.0 + jnp.exp(-x))


@functools.partial(jax.jit, static_argnums=(3, 4, 5))
def _run(params, idx_t, dns_t, num_workers, vpad, ndense):
    num_fields, B = idx_t.shape
    bpw = B // num_workers
    ngroups = bpw // _L
    psize = params.shape[0]
    mesh = plsc.VectorSubcoreMesh(core_axis_name="c", subcore_axis_name="s")

    @functools.partial(
        pl.kernel,
        mesh=mesh,
        out_type=jax.ShapeDtypeStruct((B,), jnp.float32),
        scratch_types=[
            pltpu.VMEM((psize,), jnp.float32),
            pltpu.VMEM((num_fields, bpw), jnp.int32),
            pltpu.VMEM((ndense, bpw), jnp.float32),
            pltpu.VMEM((bpw,), jnp.float32),
            pltpu.SemaphoreType.DMA,
        ],
        compiler_params=pltpu.CompilerParams(
            needs_layout_passes=False, use_tc_tiling_on_sc=True,
            disable_bounds_checks=True, disable_semaphore_checks=True
        ),
    )
    def k(params_hbm, idx_hbm, dns_hbm, out_hbm,
          params_v, idx_v, dns_v, out_v, sem):
        wid = lax.axis_index("s") * 2 + lax.axis_index("c")
        base = wid * bpw
        c1 = pltpu.async_copy(params_hbm, params_v, sem)
        c2 = pltpu.async_copy(idx_hbm.at[:, pl.ds(base, bpw)], idx_v, sem)
        c3 = pltpu.async_copy(dns_hbm.at[:, pl.ds(base, bpw)], dns_v, sem)
        c1.wait()
        c2.wait()
        c3.wait()

        lane = lax.iota(jnp.int32, _L)
        zero = jnp.zeros((_L,), jnp.int32)

        @plsc.parallel_loop(0, ngroups, 1, unroll=2)
        def body(g):
            rows = g * _L + lane
            acc = params_v[pl.ds(vpad + ndense * _L, _L)]
            for j in range(ndense):
                dv = plsc.load_gather(dns_v, [zero + j, rows])
                acc = acc + dv * params_v[pl.ds(vpad + j * _L, _L)]
            for f in range(num_fields):
                ii = plsc.load_gather(idx_v, [zero + f, rows])
                acc = acc + plsc.load_gather(params_v, [ii])
            out_v[pl.ds(g * _L, _L)] = _sigmoid(acc)
        pltpu.sync_copy(out_v, out_hbm.at[pl.ds(base, bpw)])

    return k(params, idx_t, dns_t)


def kernel(dense_input, sparse_input, emb_table, fm_W, fm_b):
    B, ndense = dense_input.shape
    V = emb_table.shape[0]
    NW = 32

    idx_t = sparse_input.astype(jnp.int32).T
    dns_t = dense_input.astype(jnp.float32).T
    vpad = ((V + 127) // 128) * 128
    params = jnp.concatenate([
        emb_table.astype(jnp.float32).reshape(V),
        jnp.zeros((vpad - V,), jnp.float32),
        jnp.broadcast_to(
            fm_W.astype(jnp.float32).reshape(ndense, 1), (ndense, _L)
        ).reshape(ndense * _L),
        jnp.broadcast_to(fm_b.astype(jnp.float32).reshape(1), (_L,)),
    ])

    out = _run(params, idx_t, dns_t, NW, vpad, ndense)
    return out.reshape(B, 1)


# speedup vs baseline: 1.0248x; 1.0028x over previous
import functools

import jax
import jax.numpy as jnp
from jax import lax
from jax.experimental import pallas as pl
from jax.experimental.pallas import tpu as pltpu
from jax.experimental.pallas import tpu_sc as plsc

_L = 16


def _sigmoid(x):
    return 1.0 / (1.0 + jnp.exp(-x))


@functools.partial(jax.jit, static_argnums=(3, 4, 5))
def _run(params, idx_t, dns_t, num_workers, vpad, ndense):
    num_fields, B = idx_t.shape
    bpw = B // num_workers
    ngroups = bpw // _L
    psize = params.shape[0]
    mesh = plsc.VectorSubcoreMesh(core_axis_name="c", subcore_axis_name="s")

    @functools.partial(
        pl.kernel,
        mesh=mesh,
        out_type=jax.ShapeDtypeStruct((B,), jnp.float32),
        scratch_types=[
            pltpu.VMEM((psize,), jnp.float32),
            pltpu.VMEM((num_fields, bpw), jnp.int32),
            pltpu.VMEM((ndense, bpw), jnp.float32),
            pltpu.VMEM((bpw,), jnp.float32),
            pltpu.SemaphoreType.DMA,
        ],
        compiler_params=pltpu.CompilerParams(
            needs_layout_passes=False, use_tc_tiling_on_sc=True
        ),
    )
    def k(params_hbm, idx_hbm, dns_hbm, out_hbm,
          params_v, idx_v, dns_v, out_v, sem):
        wid = lax.axis_index("s") * 2 + lax.axis_index("c")
        base = wid * bpw
        c1 = pltpu.async_copy(params_hbm, params_v, sem)
        c2 = pltpu.async_copy(idx_hbm.at[:, pl.ds(base, bpw)], idx_v, sem)
        c3 = pltpu.async_copy(dns_hbm.at[:, pl.ds(base, bpw)], dns_v, sem)
        c1.wait()
        c2.wait()
        c3.wait()

        lane = lax.iota(jnp.int32, _L)
        zero = jnp.zeros((_L,), jnp.int32)

        @plsc.parallel_loop(0, ngroups, 1, unroll=2)
        def body(g):
            rows = g * _L + lane
            acc = params_v[pl.ds(vpad + ndense * _L, _L)]
            for j in range(ndense):
                dv = plsc.load_gather(dns_v, [zero + j, rows])
                acc = acc + dv * params_v[pl.ds(vpad + j * _L, _L)]
            for f in range(num_fields):
                ii = plsc.load_gather(idx_v, [zero + f, rows])
                acc = acc + plsc.load_gather(params_v, [ii])
            out_v[pl.ds(g * _L, _L)] = _sigmoid(acc)
        pltpu.sync_copy(out_v, out_hbm.at[pl.ds(base, bpw)])

    return k(params, idx_t, dns_t)


def kernel(dense_input, sparse_input, emb_table, fm_W, fm_b):
    B, ndense = dense_input.shape
    V = emb_table.shape[0]
    NW = 32

    idx_t = sparse_input.astype(jnp.int32).T
    dns_t = dense_input.astype(jnp.float32).T
    vpad = ((V + 127) // 128) * 128
    params = jnp.concatenate([
        emb_table.astype(jnp.float32).reshape(V),
        jnp.zeros((vpad - V,), jnp.float32),
        jnp.broadcast_to(
            fm_W.astype(jnp.float32).reshape(ndense, 1), (ndense, _L)
        ).reshape(ndense * _L),
        jnp.broadcast_to(fm_b.astype(jnp.float32).reshape(1), (_L,)),
    ])

    out = _run(params, idx_t, dns_t, NW, vpad, ndense)
    return out.reshape(B, 1)
